# baseline (speedup 1.0000x reference)
import jax
import jax.numpy as jnp
from jax import lax
from jax.experimental import pallas as pl
from jax.experimental.pallas import tpu as pltpu
from jax.experimental.pallas import tpu_sc as plsc

H = 32
W = 32
HALF = 128
EMBED = 2 * HALF
P = H * W
B = 32
NS = 16
CPT = HALF // NS
L = 16


def _sc_body(rows_hbm, cols_hbm, out_hbm, table_v, slice_v, half_sh, sem):
    cid = lax.axis_index("c")
    sid = lax.axis_index("s")
    lanes = lax.iota(jnp.int32, L)

    @pl.when(cid == 0)
    def _build_top():
        pltpu.sync_copy(cols_hbm, table_v)
        for k in range(CPT):
            c = jnp.full((L,), CPT * sid + k, jnp.int32)
            g0 = plsc.load_gather(table_v, [lanes, c])
            g1 = plsc.load_gather(table_v, [lanes + L, c])
            for m in range(W):
                slice_v[k, pl.ds(2 * L * m, L)] = g0
                slice_v[k, pl.ds(2 * L * m + L, L)] = g1

    @pl.when(cid == 1)
    def _build_bottom():
        pltpu.sync_copy(rows_hbm, table_v)
        for k in range(CPT):
            c = jnp.full((L,), CPT * sid + k, jnp.int32)
            for y in range(H):
                g = plsc.load_gather(table_v, [jnp.full((L,), y, jnp.int32), c])
                slice_v[k, pl.ds(2 * L * y, L)] = g
                slice_v[k, pl.ds(2 * L * y + L, L)] = g

    pltpu.sync_copy(slice_v, half_sh.at[pl.ds(CPT * sid, CPT), :])
    plsc.subcore_barrier()

    @pl.when(cid == 0)
    def _write_top():
        for q in range(2):
            b = 2 * sid + q
            pltpu.make_async_copy(
                half_sh, out_hbm.at[b, pl.ds(0, HALF), :], sem).start()
        for q in range(2):
            b = 2 * sid + q
            pltpu.make_async_copy(
                half_sh, out_hbm.at[b, pl.ds(0, HALF), :], sem).wait()

    @pl.when(cid == 1)
    def _write_bottom():
        for q in range(2):
            b = 2 * sid + q
            pltpu.make_async_copy(
                half_sh, out_hbm.at[b, pl.ds(HALF, HALF), :], sem).start()
        for q in range(2):
            b = 2 * sid + q
            pltpu.make_async_copy(
                half_sh, out_hbm.at[b, pl.ds(HALF, HALF), :], sem).wait()


def kernel(pixel_values, rows_emb, cols_emb):
    b = pixel_values.shape[0]
    mesh = plsc.VectorSubcoreMesh(core_axis_name="c", subcore_axis_name="s",
                                  num_cores=2, num_subcores=NS)
    k = pl.kernel(
        _sc_body,
        out_type=jax.ShapeDtypeStruct((b, EMBED, P), jnp.float32),
        mesh=mesh,
        scratch_types=[
            pltpu.VMEM((50, HALF), jnp.float32),
            pltpu.VMEM((CPT, P), jnp.float32),
            pltpu.VMEM_SHARED((HALF, P), jnp.float32),
            pltpu.SemaphoreType.DMA,
        ],
        compiler_params=pltpu.CompilerParams(needs_layout_passes=False),
    )
    out = k(rows_emb, cols_emb)
    return out.reshape(b, EMBED, H, W)
